# baseline (speedup 1.0000x reference)
import functools

import jax
import jax.numpy as jnp
from jax import lax
from jax.experimental import pallas as pl
from jax.experimental.pallas import tpu as pltpu
from jax.experimental.pallas import tpu_sc as plsc

NUM_LOC = 1000000
EMBED_SIZE = 64
BATCH = 4096
HIST_LEN = 200

NC = 2
NS = 16
NW = NC * NS
TOTAL = BATCH * HIST_LEN
PER_W = TOTAL // NW
CHUNK = 128
NCHUNK = PER_W // CHUNK
K = 8
NGROUP = NCHUNK // K


def _body(tok_hbm, embed_hbm, out_hbm, idx_v, rows_v, gsem, ssem):
    wid = lax.axis_index("s") * NC + lax.axis_index("c")
    base = wid * PER_W

    pltpu.sync_copy(tok_hbm.at[wid], idx_v)

    def group(t, _):
        gathers = []
        for b in range(K):
            j = t * K + b
            gathers.append(
                pltpu.async_copy(embed_hbm.at[idx_v.at[j]], rows_v.at[b], gsem))
        for cp in gathers:
            cp.wait()
        scatters = []
        for b in range(K):
            j = t * K + b
            scatters.append(
                pltpu.async_copy(
                    rows_v.at[b], out_hbm.at[pl.ds(base + j * CHUNK, CHUNK)],
                    ssem))
        for cp in scatters:
            cp.wait()
        return _

    lax.fori_loop(0, NGROUP, group, None)


@jax.jit
def _embed_lookup(tok3, embed):
    mesh = plsc.VectorSubcoreMesh(core_axis_name="c", subcore_axis_name="s")
    run = functools.partial(
        pl.kernel,
        out_type=jax.ShapeDtypeStruct((TOTAL, EMBED_SIZE), jnp.float32),
        mesh=mesh,
        scratch_types=[
            pltpu.VMEM((NCHUNK, CHUNK), jnp.int32),
            pltpu.VMEM((K, CHUNK, EMBED_SIZE), jnp.float32),
            pltpu.SemaphoreType.DMA,
            pltpu.SemaphoreType.DMA,
        ],
        compiler_params=pltpu.CompilerParams(use_tc_tiling_on_sc=False),
    )(_body)
    return run(tok3, embed)


def kernel(token, embed):
    tok3 = token.reshape(NW, NCHUNK, CHUNK).astype(jnp.int32)
    out = _embed_lookup(tok3, embed)
    return out.reshape(BATCH, HIST_LEN, EMBED_SIZE)


# speedup vs baseline: 1.0045x; 1.0045x over previous
import functools

import jax
import jax.numpy as jnp
from jax import lax
from jax.experimental import pallas as pl
from jax.experimental.pallas import tpu as pltpu
from jax.experimental.pallas import tpu_sc as plsc

NUM_LOC = 1000000
EMBED_SIZE = 64
BATCH = 4096
HIST_LEN = 200

NC = 2
NS = 16
NW = NC * NS
TOTAL = BATCH * HIST_LEN
PER_W = TOTAL // NW
CHUNK = 128
NCHUNK = PER_W // CHUNK
K = 4
NGROUP = NCHUNK // K


def _body(tok_hbm, embed_hbm, out_hbm, idx_v, rows_v,
          gsem0, gsem1, ssem0, ssem1):
    wid = lax.axis_index("s") * NC + lax.axis_index("c")
    base = wid * PER_W

    pltpu.sync_copy(tok_hbm.at[wid], idx_v)

    def fire_gathers(grp, bset, sem):
        cps = []
        for b in range(K):
            cps.append(pltpu.async_copy(
                embed_hbm.at[idx_v.at[grp * K + b]], rows_v.at[bset, b], sem))
        return cps

    def fire_scatters(grp, bset, sem):
        cps = []
        for b in range(K):
            cps.append(pltpu.async_copy(
                rows_v.at[bset, b],
                out_hbm.at[pl.ds(base + (grp * K + b) * CHUNK, CHUNK)], sem))
        return cps

    def drain(grp, bset, sem, out=False):
        for b in range(K):
            if out:
                cp = pltpu.make_async_copy(
                    rows_v.at[bset, b],
                    out_hbm.at[pl.ds(base + (grp * K + b) * CHUNK, CHUNK)],
                    sem)
            else:
                cp = pltpu.make_async_copy(
                    embed_hbm.at[idx_v.at[grp * K + b]], rows_v.at[bset, b],
                    sem)
            cp.wait()

    fire_gathers(0, 0, gsem0)

    def pair(u, _):
        g_even, g_odd = 2 * u, 2 * u + 1
        drain(g_even, 0, gsem0)
        fire_gathers(g_odd, 1, gsem1)
        fire_scatters(g_even, 0, ssem0)
        drain(g_odd, 1, gsem1)
        drain(g_even, 0, ssem0, out=True)

        @pl.when(u + 1 < NGROUP // 2)
        def _():
            fire_gathers(g_even + 2, 0, gsem0)

        fire_scatters(g_odd, 1, ssem1)
        drain(g_odd, 1, ssem1, out=True)
        return _

    lax.fori_loop(0, NGROUP // 2, pair, None)


@jax.jit
def _embed_lookup(tok3, embed):
    mesh = plsc.VectorSubcoreMesh(core_axis_name="c", subcore_axis_name="s")
    run = functools.partial(
        pl.kernel,
        out_type=jax.ShapeDtypeStruct((TOTAL, EMBED_SIZE), jnp.float32),
        mesh=mesh,
        scratch_types=[
            pltpu.VMEM((NCHUNK, CHUNK), jnp.int32),
            pltpu.VMEM((2, K, CHUNK, EMBED_SIZE), jnp.float32),
            pltpu.SemaphoreType.DMA,
            pltpu.SemaphoreType.DMA,
            pltpu.SemaphoreType.DMA,
            pltpu.SemaphoreType.DMA,
        ],
        compiler_params=pltpu.CompilerParams(use_tc_tiling_on_sc=False),
    )(_body)
    return run(tok3, embed)


def kernel(token, embed):
    tok3 = token.reshape(NW, NCHUNK, CHUNK).astype(jnp.int32)
    out = _embed_lookup(tok3, embed)
    return out.reshape(BATCH, HIST_LEN, EMBED_SIZE)


# speedup vs baseline: 1.0103x; 1.0058x over previous
import functools

import jax
import jax.numpy as jnp
from jax import lax
from jax.experimental import pallas as pl
from jax.experimental.pallas import tpu as pltpu
from jax.experimental.pallas import tpu_sc as plsc

NUM_LOC = 1000000
EMBED_SIZE = 64
BATCH = 4096
HIST_LEN = 200

NC = 2
NS = 16
NW = NC * NS
ROWS_W = BATCH // NW
CA = 104
CB = HIST_LEN - CA
NSLOT = 4


def _body(tok_hbm, embed_hbm, out_hbm, idx_v, rows_v,
          g0, g1, g2, g3, s0, s1, s2, s3):
    wid = lax.axis_index("s") * NC + lax.axis_index("c")
    rbase = wid * ROWS_W
    gsems = (g0, g1, g2, g3)
    ssems = (s0, s1, s2, s3)

    pltpu.sync_copy(tok_hbm.at[pl.ds(rbase, ROWS_W)], idx_v)

    def gather_cps(r, slot, sem):
        return (
            pltpu.make_async_copy(
                embed_hbm.at[idx_v.at[r, pl.ds(0, CA)]],
                rows_v.at[slot, pl.ds(0, CA)], sem),
            pltpu.make_async_copy(
                embed_hbm.at[idx_v.at[r, pl.ds(CA, CB)]],
                rows_v.at[slot, pl.ds(CA, CB)], sem),
        )

    def scatter_cp(r, slot, sem):
        return pltpu.make_async_copy(
            rows_v.at[slot], out_hbm.at[rbase + r], sem)

    def fire_gather(r, slot):
        for cp in gather_cps(r, slot, gsems[slot]):
            cp.start()

    def wait_gather(r, slot):
        for cp in gather_cps(r, slot, gsems[slot]):
            cp.wait()

    fire_gather(0, 0)
    fire_gather(1, 1)

    def step(t, _):
        for si in range(NSLOT):
            r = NSLOT * t + si
            wait_gather(r, si)
            scatter_cp(r, si, ssems[si]).start()
            q = (si + 2) % NSLOT

            @pl.when(r >= 2)
            def _():
                scatter_cp(r - 2, q, ssems[q]).wait()

            @pl.when(r + 2 < ROWS_W)
            def _():
                fire_gather(r + 2, q)
        return _

    lax.fori_loop(0, ROWS_W // NSLOT, step, None)

    scatter_cp(ROWS_W - 2, (ROWS_W - 2) % NSLOT,
               ssems[(ROWS_W - 2) % NSLOT]).wait()
    scatter_cp(ROWS_W - 1, (ROWS_W - 1) % NSLOT,
               ssems[(ROWS_W - 1) % NSLOT]).wait()


@jax.jit
def _embed_lookup(token, embed):
    mesh = plsc.VectorSubcoreMesh(core_axis_name="c", subcore_axis_name="s")
    run = functools.partial(
        pl.kernel,
        out_type=jax.ShapeDtypeStruct((BATCH, HIST_LEN, EMBED_SIZE),
                                      jnp.float32),
        mesh=mesh,
        scratch_types=[
            pltpu.VMEM((ROWS_W, HIST_LEN), jnp.int32),
            pltpu.VMEM((NSLOT, HIST_LEN, EMBED_SIZE), jnp.float32),
            pltpu.SemaphoreType.DMA,
            pltpu.SemaphoreType.DMA,
            pltpu.SemaphoreType.DMA,
            pltpu.SemaphoreType.DMA,
            pltpu.SemaphoreType.DMA,
            pltpu.SemaphoreType.DMA,
            pltpu.SemaphoreType.DMA,
            pltpu.SemaphoreType.DMA,
        ],
        compiler_params=pltpu.CompilerParams(use_tc_tiling_on_sc=False),
    )(_body)
    return run(token, embed)


def kernel(token, embed):
    return _embed_lookup(token.astype(jnp.int32), embed)


# speedup vs baseline: 1.3404x; 1.3267x over previous
import functools

import jax
import jax.numpy as jnp
from jax import lax
from jax.experimental import pallas as pl
from jax.experimental.pallas import tpu as pltpu
from jax.experimental.pallas import tpu_sc as plsc

NUM_LOC = 1000000
EMBED_SIZE = 64
BATCH = 4096
HIST_LEN = 200

NC = 2
NS = 16
NW = NC * NS
ROWS_W = BATCH // NW
CA = 104
CB = HIST_LEN - CA
NSLOT = 4


def _body(tok_hbm, embed_hbm, out_hbm, idx_v, rows_v,
          g0, g1, g2, g3, s0, s1, s2, s3):
    wid = lax.axis_index("s") * NC + lax.axis_index("c")
    rbase = wid * ROWS_W
    gsems = (g0, g1, g2, g3)
    ssems = (s0, s1, s2, s3)

    pltpu.sync_copy(tok_hbm.at[pl.ds(rbase, ROWS_W)], idx_v)

    def gather_cps(r, slot, sem):
        return (
            pltpu.make_async_copy(
                embed_hbm.at[idx_v.at[r, pl.ds(0, CA)]],
                rows_v.at[slot, pl.ds(0, CA)], sem),
            pltpu.make_async_copy(
                embed_hbm.at[idx_v.at[r, pl.ds(CA, CB)]],
                rows_v.at[slot, pl.ds(CA, CB)], sem),
        )

    def scatter_cp(r, slot, sem):
        return pltpu.make_async_copy(
            rows_v.at[slot],
            out_hbm.at[pl.ds((rbase + r) * HIST_LEN, HIST_LEN),
                       pl.ds(0, EMBED_SIZE)], sem)

    def fire_gather(r, slot):
        for cp in gather_cps(r, slot, gsems[slot]):
            cp.start()

    def wait_gather(r, slot):
        for cp in gather_cps(r, slot, gsems[slot]):
            cp.wait()

    fire_gather(0, 0)
    fire_gather(1, 1)

    def step(t, _):
        for si in range(NSLOT):
            r = NSLOT * t + si
            wait_gather(r, si)
            scatter_cp(r, si, ssems[si]).start()
            q = (si + 2) % NSLOT

            @pl.when(r >= 2)
            def _():
                scatter_cp(r - 2, q, ssems[q]).wait()

            @pl.when(r + 2 < ROWS_W)
            def _():
                fire_gather(r + 2, q)
        return _

    lax.fori_loop(0, ROWS_W // NSLOT, step, None)

    scatter_cp(ROWS_W - 2, (ROWS_W - 2) % NSLOT,
               ssems[(ROWS_W - 2) % NSLOT]).wait()
    scatter_cp(ROWS_W - 1, (ROWS_W - 1) % NSLOT,
               ssems[(ROWS_W - 1) % NSLOT]).wait()


@jax.jit
def _embed_lookup(token, embed):
    mesh = plsc.VectorSubcoreMesh(core_axis_name="c", subcore_axis_name="s")
    run = functools.partial(
        pl.kernel,
        out_type=jax.ShapeDtypeStruct((BATCH * HIST_LEN, 128), jnp.float32),
        mesh=mesh,
        scratch_types=[
            pltpu.VMEM((ROWS_W, HIST_LEN), jnp.int32),
            pltpu.VMEM((NSLOT, HIST_LEN, EMBED_SIZE), jnp.float32),
            pltpu.SemaphoreType.DMA,
            pltpu.SemaphoreType.DMA,
            pltpu.SemaphoreType.DMA,
            pltpu.SemaphoreType.DMA,
            pltpu.SemaphoreType.DMA,
            pltpu.SemaphoreType.DMA,
            pltpu.SemaphoreType.DMA,
            pltpu.SemaphoreType.DMA,
        ],
        compiler_params=pltpu.CompilerParams(use_tc_tiling_on_sc=False),
    )(_body)
    return run(token, embed)


def kernel(token, embed):
    out = _embed_lookup(token.astype(jnp.int32), embed)
    return out[:, :EMBED_SIZE].reshape(BATCH, HIST_LEN, EMBED_SIZE)
